# baseline (speedup 1.0000x reference)
import functools

import jax
import jax.numpy as jnp
from jax import lax
from jax.experimental import pallas as pl
from jax.experimental.pallas import tpu as pltpu
from jax.experimental.pallas import tpu_sc as plsc

NC, NS, LANES = 2, 16, 16
NW = NC * NS
B, L, E = 4096, 50, 64
BPW = B // NW
EC = E // LANES
CR = 4
NCHUNK = BPW // CR
VOCAB = 100000
TBLK = 7168
VPAD = 100352


def _fmt_body(tt_ref, out_ref):
    out_ref[:, :E] = tt_ref[...].T.astype(jnp.bfloat16)


@jax.jit
def _fmt_table(tt):
    return pl.pallas_call(
        _fmt_body,
        grid=(VPAD // TBLK,),
        in_specs=[pl.BlockSpec((E, TBLK), lambda i: (0, i))],
        out_specs=pl.BlockSpec((TBLK, 128), lambda i: (i, 0)),
        out_shape=jax.ShapeDtypeStruct((VPAD, 128), jnp.bfloat16),
    )(tt)


def _sc_embed_body(table_hbm, x_hbm, w_hbm, emb_hbm,
                   idx_v, w_v, rows0_v, rows1_v, out_v, sem0, sem1):
    wid = lax.axis_index("s") * NC + lax.axis_index("c")
    base = wid * (BPW * L)

    pltpu.sync_copy(x_hbm.at[pl.ds(base, BPW * L)], idx_v)
    pltpu.sync_copy(w_hbm.at[pl.ds(base, BPW * L)], w_v)

    def gstart(j, buf, sem):
        pltpu.async_copy(
            table_hbm.at[idx_v.at[pl.ds(j * (CR * L), CR * L)]], buf, sem
        )

    def gwait(j, buf, sem):
        pltpu.make_async_copy(
            table_hbm.at[idx_v.at[pl.ds(j * (CR * L), CR * L)]], buf, sem
        ).wait()

    def accum(j, buf):
        for r8 in range(CR):
            flat0 = r8 * L

            def lstep(t, accs):
                wv = plsc.load_gather(
                    w_v, [lax.broadcast(j * (CR * L) + flat0 + t, (LANES,))]
                )
                new = []
                for h in range(2):
                    seg = buf[flat0 + t, pl.ds(h * 32, 32)]
                    lo, hi = plsc.unpack(seg, format=plsc.PackFormat.INTERLEAVED)
                    new.append(accs[2 * h] + lo * wv)
                    new.append(accs[2 * h + 1] + hi * wv)
                return tuple(new)

            def lbody(i, accs):
                for u in range(5):
                    accs = lstep(5 * i + u, accs)
                return accs

            zero = jnp.zeros((LANES,), jnp.float32)
            accs = lax.fori_loop(0, L // 5, lbody, (zero,) * EC)
            for c in range(EC):
                out_v[j * CR + r8, pl.ds(c * LANES, LANES)] = accs[c]

    gstart(0, rows0_v, sem0)
    gstart(1, rows1_v, sem1)

    @pl.loop(0, NCHUNK, step=2)
    def _(j):
        gwait(j, rows0_v, sem0)

        @pl.when(j + 2 < NCHUNK)
        def _():
            gstart(j + 2, rows0_v, sem0)

        accum(j, rows0_v)
        gwait(j + 1, rows1_v, sem1)

        @pl.when(j + 3 < NCHUNK)
        def _():
            gstart(j + 3, rows1_v, sem1)

        accum(j + 1, rows1_v)

    pltpu.sync_copy(out_v, emb_hbm.at[pl.ds(wid * BPW, BPW)])


@jax.jit
def _sc_embed(table, xf, wf):
    mesh = plsc.VectorSubcoreMesh(core_axis_name="c", subcore_axis_name="s")
    kern = pl.kernel(
        _sc_embed_body,
        out_type=jax.ShapeDtypeStruct((B, E), jnp.float32),
        mesh=mesh,
        compiler_params=pltpu.CompilerParams(
            needs_layout_passes=False, use_tc_tiling_on_sc=False
        ),
        scratch_types=[
            pltpu.VMEM((BPW * L,), jnp.int32),
            pltpu.VMEM((BPW * L,), jnp.float32),
            pltpu.VMEM((CR * L, E), jnp.bfloat16),
            pltpu.VMEM((CR * L, E), jnp.bfloat16),
            pltpu.VMEM((BPW, E), jnp.float32),
            pltpu.SemaphoreType.DMA,
            pltpu.SemaphoreType.DMA,
        ],
    )
    return kern(table, xf, wf)


def _mlp_body(emb_ref, w1t_ref, b1_ref, w2t_ref, b2_ref, out_ref):
    h = jnp.dot(emb_ref[...], w1t_ref[...],
                preferred_element_type=jnp.float32) + b1_ref[...]
    h = jnp.maximum(h, 0.0)
    out_ref[...] = jnp.dot(h, w2t_ref[...],
                           preferred_element_type=jnp.float32) + b2_ref[...]


@jax.jit
def _mlp(emb, w1t, b1, w2t, b2):
    nblk = 4
    blk = B // nblk
    return pl.pallas_call(
        _mlp_body,
        grid=(nblk,),
        in_specs=[
            pl.BlockSpec((blk, E), lambda i: (i, 0)),
            pl.BlockSpec((E, 128), lambda i: (0, 0)),
            pl.BlockSpec((1, 128), lambda i: (0, 0)),
            pl.BlockSpec((128, E), lambda i: (0, 0)),
            pl.BlockSpec((1, E), lambda i: (0, 0)),
        ],
        out_specs=pl.BlockSpec((blk, E), lambda i: (i, 0)),
        out_shape=jax.ShapeDtypeStruct((B, E), jnp.float32),
    )(emb, w1t, b1, w2t, b2)


def kernel(x, w, table, W1, b1, W2, b2):
    xf = (x.reshape(-1) * 2).astype(jnp.int32)
    wf = w.reshape(-1).astype(jnp.float32)
    tlin = _fmt_table(table.T).reshape(2 * VPAD, E)
    emb = _sc_embed(tlin, xf, wf)
    perm = jnp.arange(64).reshape(2, 16, 2).transpose(0, 2, 1).reshape(-1)
    return _mlp(emb, W1.T[perm], b1.reshape(1, -1), W2.T, b2.reshape(1, -1))


# speedup vs baseline: 1.8334x; 1.8334x over previous
import functools

import jax
import jax.numpy as jnp
from jax import lax
from jax.experimental import pallas as pl
from jax.experimental.pallas import tpu as pltpu
from jax.experimental.pallas import tpu_sc as plsc

NC, NS, LANES = 2, 16, 16
NW = NC * NS
B, L, E = 4096, 50, 64
BPW = B // NW
EC = E // LANES
CR = 4
NCHUNK = BPW // CR
VOCAB = 100000
TBLK = 7168
VPAD = 100352


def _fmt_body(tt_ref, out_ref):
    out_ref[:, :E] = tt_ref[...].T


@jax.jit
def _fmt_table(tt):
    return pl.pallas_call(
        _fmt_body,
        grid=(VPAD // TBLK,),
        in_specs=[pl.BlockSpec((E, TBLK), lambda i: (0, i))],
        out_specs=pl.BlockSpec((TBLK, 128), lambda i: (i, 0)),
        out_shape=jax.ShapeDtypeStruct((VPAD, 128), jnp.float32),
    )(tt)


def _sc_embed_body(table_hbm, x_hbm, w_hbm, emb_hbm,
                   idx_v, w_v, rows0_v, rows1_v, out_v, sem0, sem1):
    wid = lax.axis_index("s") * NC + lax.axis_index("c")
    base = wid * (BPW * L)

    pltpu.sync_copy(x_hbm.at[pl.ds(base, BPW * L)], idx_v)
    pltpu.sync_copy(w_hbm.at[pl.ds(base, BPW * L)], w_v)

    def gstart(j, buf, sem):
        pltpu.async_copy(
            table_hbm.at[idx_v.at[pl.ds(j * (CR * L), CR * L)]], buf, sem
        )

    def gwait(j, buf, sem):
        pltpu.make_async_copy(
            table_hbm.at[idx_v.at[pl.ds(j * (CR * L), CR * L)]], buf, sem
        ).wait()

    def accum(j, buf):
        for r8 in range(CR):
            flat0 = r8 * L

            def lstep(t, accs):
                wv = plsc.load_gather(
                    w_v, [lax.broadcast(j * (CR * L) + flat0 + t, (LANES,))]
                )
                return tuple(
                    accs[c] + buf[flat0 + t, pl.ds(c * LANES, LANES)] * wv
                    for c in range(EC)
                )

            def lbody(i, accs):
                for u in range(5):
                    accs = lstep(5 * i + u, accs)
                return accs

            zero = jnp.zeros((LANES,), jnp.float32)
            accs = lax.fori_loop(0, L // 5, lbody, (zero,) * EC)
            for c in range(EC):
                out_v[j * CR + r8, pl.ds(c * LANES, LANES)] = accs[c]

    gstart(0, rows0_v, sem0)
    gstart(1, rows1_v, sem1)

    @pl.loop(0, NCHUNK, step=2)
    def _(j):
        gwait(j, rows0_v, sem0)

        @pl.when(j + 2 < NCHUNK)
        def _():
            gstart(j + 2, rows0_v, sem0)

        accum(j, rows0_v)
        gwait(j + 1, rows1_v, sem1)

        @pl.when(j + 3 < NCHUNK)
        def _():
            gstart(j + 3, rows1_v, sem1)

        accum(j + 1, rows1_v)

    pltpu.sync_copy(out_v, emb_hbm.at[pl.ds(wid * BPW, BPW)])


@jax.jit
def _sc_embed(table, xf, wf):
    mesh = plsc.VectorSubcoreMesh(core_axis_name="c", subcore_axis_name="s")
    kern = pl.kernel(
        _sc_embed_body,
        out_type=jax.ShapeDtypeStruct((B, E), jnp.float32),
        mesh=mesh,
        compiler_params=pltpu.CompilerParams(
            needs_layout_passes=False, use_tc_tiling_on_sc=False
        ),
        scratch_types=[
            pltpu.VMEM((BPW * L,), jnp.int32),
            pltpu.VMEM((BPW * L,), jnp.float32),
            pltpu.VMEM((CR * L, E), jnp.float32),
            pltpu.VMEM((CR * L, E), jnp.float32),
            pltpu.VMEM((BPW, E), jnp.float32),
            pltpu.SemaphoreType.DMA,
            pltpu.SemaphoreType.DMA,
        ],
    )
    return kern(table, xf, wf)


def _mlp_body(emb_ref, w1t_ref, b1_ref, w2t_ref, b2_ref, out_ref):
    h = jnp.dot(emb_ref[...], w1t_ref[...],
                preferred_element_type=jnp.float32) + b1_ref[...]
    h = jnp.maximum(h, 0.0)
    out_ref[...] = jnp.dot(h, w2t_ref[...],
                           preferred_element_type=jnp.float32) + b2_ref[...]


@jax.jit
def _mlp(emb, w1t, b1, w2t, b2):
    nblk = 4
    blk = B // nblk
    return pl.pallas_call(
        _mlp_body,
        grid=(nblk,),
        in_specs=[
            pl.BlockSpec((blk, E), lambda i: (i, 0)),
            pl.BlockSpec((E, 128), lambda i: (0, 0)),
            pl.BlockSpec((1, 128), lambda i: (0, 0)),
            pl.BlockSpec((128, E), lambda i: (0, 0)),
            pl.BlockSpec((1, E), lambda i: (0, 0)),
        ],
        out_specs=pl.BlockSpec((blk, E), lambda i: (i, 0)),
        out_shape=jax.ShapeDtypeStruct((B, E), jnp.float32),
    )(emb, w1t, b1, w2t, b2)


def kernel(x, w, table, W1, b1, W2, b2):
    xf = (x.reshape(-1) * 2).astype(jnp.int32)
    wf = w.reshape(-1).astype(jnp.float32)
    tlin = _fmt_table(table.T).reshape(2 * VPAD, E)
    emb = _sc_embed(tlin, xf, wf)
    return _mlp(emb, W1.T, b1.reshape(1, -1), W2.T, b2.reshape(1, -1))


# speedup vs baseline: 1.8750x; 1.0227x over previous
import functools

import jax
import jax.numpy as jnp
from jax import lax
from jax.experimental import pallas as pl
from jax.experimental.pallas import tpu as pltpu
from jax.experimental.pallas import tpu_sc as plsc

NC, NS, LANES = 2, 16, 16
NW = NC * NS
B, L, E = 4096, 50, 64
BPW = B // NW
EC = E // LANES
CR = 4
NCHUNK = BPW // CR
VOCAB = 100000
TBLK = 14336
VPAD = 100352


def _fmt_body(tt_ref, out_ref):
    out_ref[:, :E] = tt_ref[...].T


@jax.jit
def _fmt_table(tt):
    return pl.pallas_call(
        _fmt_body,
        grid=(VPAD // TBLK,),
        in_specs=[pl.BlockSpec((E, TBLK), lambda i: (0, i))],
        out_specs=pl.BlockSpec((TBLK, 128), lambda i: (i, 0)),
        out_shape=jax.ShapeDtypeStruct((VPAD, 128), jnp.float32),
    )(tt)


def _sc_embed_body(table_hbm, x_hbm, w_hbm, emb_hbm,
                   idx_v, w_v, rows0_v, rows1_v, out_v, sem0, sem1):
    wid = lax.axis_index("s") * NC + lax.axis_index("c")
    base = wid * (BPW * L)

    pltpu.sync_copy(x_hbm.at[pl.ds(base, BPW * L)], idx_v)
    pltpu.sync_copy(w_hbm.at[pl.ds(base, BPW * L)], w_v)

    def gstart(j, buf, sem):
        pltpu.async_copy(
            table_hbm.at[idx_v.at[pl.ds(j * (CR * L), CR * L)]], buf, sem
        )

    def gwait(j, buf, sem):
        pltpu.make_async_copy(
            table_hbm.at[idx_v.at[pl.ds(j * (CR * L), CR * L)]], buf, sem
        ).wait()

    def accum(j, buf):
        for r8 in range(CR):
            flat0 = r8 * L

            def lstep(t, accs):
                wv = plsc.load_gather(
                    w_v, [lax.broadcast(j * (CR * L) + flat0 + t, (LANES,))]
                )
                return tuple(
                    accs[c] + buf[flat0 + t, pl.ds(c * LANES, LANES)] * wv
                    for c in range(EC)
                )

            def lbody(i, accs):
                for u in range(5):
                    accs = lstep(5 * i + u, accs)
                return accs

            zero = jnp.zeros((LANES,), jnp.float32)
            accs = lax.fori_loop(0, L // 5, lbody, (zero,) * EC)
            for c in range(EC):
                out_v[j * CR + r8, pl.ds(c * LANES, LANES)] = accs[c]

    gstart(0, rows0_v, sem0)
    gstart(1, rows1_v, sem1)

    @pl.loop(0, NCHUNK, step=2)
    def _(j):
        gwait(j, rows0_v, sem0)

        @pl.when(j + 2 < NCHUNK)
        def _():
            gstart(j + 2, rows0_v, sem0)

        accum(j, rows0_v)
        gwait(j + 1, rows1_v, sem1)

        @pl.when(j + 3 < NCHUNK)
        def _():
            gstart(j + 3, rows1_v, sem1)

        accum(j + 1, rows1_v)

    pltpu.sync_copy(out_v, emb_hbm.at[pl.ds(wid * BPW, BPW)])


@jax.jit
def _sc_embed(table, xf, wf):
    mesh = plsc.VectorSubcoreMesh(core_axis_name="c", subcore_axis_name="s")
    kern = pl.kernel(
        _sc_embed_body,
        out_type=jax.ShapeDtypeStruct((B, E), jnp.float32),
        mesh=mesh,
        compiler_params=pltpu.CompilerParams(
            needs_layout_passes=False, use_tc_tiling_on_sc=False
        ),
        scratch_types=[
            pltpu.VMEM((BPW * L,), jnp.int32),
            pltpu.VMEM((BPW * L,), jnp.float32),
            pltpu.VMEM((CR * L, E), jnp.float32),
            pltpu.VMEM((CR * L, E), jnp.float32),
            pltpu.VMEM((BPW, E), jnp.float32),
            pltpu.SemaphoreType.DMA,
            pltpu.SemaphoreType.DMA,
        ],
    )
    return kern(table, xf, wf)


def _mlp_body(emb_ref, w1t_ref, b1_ref, w2t_ref, b2_ref, out_ref):
    h = jnp.dot(emb_ref[...], w1t_ref[...],
                preferred_element_type=jnp.float32) + b1_ref[...]
    h = jnp.maximum(h, 0.0)
    out_ref[...] = jnp.dot(h, w2t_ref[...],
                           preferred_element_type=jnp.float32) + b2_ref[...]


@jax.jit
def _mlp(emb, w1t, b1, w2t, b2):
    nblk = 4
    blk = B // nblk
    return pl.pallas_call(
        _mlp_body,
        grid=(nblk,),
        in_specs=[
            pl.BlockSpec((blk, E), lambda i: (i, 0)),
            pl.BlockSpec((E, 128), lambda i: (0, 0)),
            pl.BlockSpec((1, 128), lambda i: (0, 0)),
            pl.BlockSpec((128, E), lambda i: (0, 0)),
            pl.BlockSpec((1, E), lambda i: (0, 0)),
        ],
        out_specs=pl.BlockSpec((blk, E), lambda i: (i, 0)),
        out_shape=jax.ShapeDtypeStruct((B, E), jnp.float32),
    )(emb, w1t, b1, w2t, b2)


def kernel(x, w, table, W1, b1, W2, b2):
    xf = (x.reshape(-1) * 2).astype(jnp.int32)
    wf = w.reshape(-1).astype(jnp.float32)
    tlin = _fmt_table(table.T).reshape(2 * VPAD, E)
    emb = _sc_embed(tlin, xf, wf)
    return _mlp(emb, W1.T, b1.reshape(1, -1), W2.T, b2.reshape(1, -1))
